# baseline (speedup 1.0000x reference)
import functools

import jax
import jax.numpy as jnp
from jax import lax
from jax.experimental import pallas as pl
from jax.experimental.pallas import tpu as pltpu
from jax.experimental.pallas import tpu_sc as plsc

N = 10000
E = 320000
NE = 5
D_OP = 128
D_T = 16
NH = 64
EH = 8
L = 6

NPAD = 10112
NW = 32
EPW = E // NW
CK = 128
NC = 80
SLOTS = NC * CK
PADN = SLOTS - EPW
RPT = NPAD // 16
EPW8 = EPW // 8
SLOTS8 = SLOTS // 8
IDXR = NE * NW * NC

_mesh = plsc.VectorSubcoreMesh(core_axis_name="c", subcore_axis_name="s")


def _elu(x):
    return jnp.where(x > 0, x, jnp.exp(jnp.minimum(x, 0.0)) - 1.0)


def _h0_body(x_ref, w_ref, b_ref, o_ref):
    o_ref[...] = _elu(
        jnp.dot(x_ref[...], w_ref[...], preferred_element_type=jnp.float32)
        + b_ref[...][None, :])


def _efsp_body(tf_ref, w_ref, b_ref, o_ref):
    o_ref[0, 0] = _elu(
        jnp.dot(tf_ref[...], w_ref[0], preferred_element_type=jnp.float32)
        + b_ref[0, 0][None, :])


def _combine_body(last, h_ref, p_ref, invc_ref, mask_ref, me_ref,
                  w1_ref, w2_ref, b_ref, o_ref):
    h = h_ref[...]
    acc = jnp.zeros_like(h)
    for i in range(NE):
        t = (p_ref[0, i] + p_ref[1, i]) * invc_ref[:, i][:, None]
        acc = acc + jnp.dot(t, w1_ref[i], preferred_element_type=jnp.float32)
        acc = acc + jnp.dot(me_ref[i], w2_ref[i], preferred_element_type=jnp.float32)
        acc = acc + mask_ref[:, i][:, None] * b_ref[i][None, :]
    r = h + acc * (1.0 / NE)
    o_ref[...] = r if last else _elu(r)


def _final_body(h_ref, w_ref, b_ref, o_ref):
    o_ref[...] = jnp.dot(h_ref[...], w_ref[...],
                         preferred_element_type=jnp.float32) + b_ref[0]


def _zero_rows(buf, rows, width):
    z = jnp.zeros((16,), jnp.float32)

    @pl.loop(0, rows)
    def _(r):
        for k in range(width // 16):
            buf[r, pl.ds(k * 16, 16)] = z


def _pass0_body(efsp, dstw, out, tab, idx, ebuf, zbuf, sem0, sem1):
    c = lax.axis_index("c")
    s = lax.axis_index("s")
    w = c * 16 + s
    _zero_rows(zbuf, RPT, 16)
    pltpu.sync_copy(zbuf, tab.at[pl.ds(s * RPT, RPT)])
    plsc.subcore_barrier()
    for i in range(NE):
        pltpu.sync_copy(dstw.at[pl.ds((i * NW + w) * NC, NC)], idx)
        pltpu.async_copy(efsp.at[i, w, pl.ds(0, CK)], ebuf.at[0], sem0)

        @pl.loop(0, NC, step=2)
        def _(ch):
            d1 = pltpu.async_copy(
                efsp.at[i, w, pl.ds((ch + 1) * CK, CK)], ebuf.at[1], sem1)
            pltpu.make_async_copy(
                efsp.at[i, w, pl.ds(ch * CK, CK)], ebuf.at[0], sem0).wait()
            pltpu.sync_copy(ebuf.at[0], tab.at[idx.at[ch]], add=True)

            @pl.when(ch + 2 < NC)
            def _():
                pltpu.async_copy(
                    efsp.at[i, w, pl.ds((ch + 2) * CK, CK)], ebuf.at[0], sem0)

            d1.wait()
            pltpu.sync_copy(ebuf.at[1], tab.at[idx.at[ch + 1]], add=True)

        plsc.subcore_barrier()
        pltpu.sync_copy(tab.at[pl.ds(s * RPT, RPT)],
                        out.at[c, i, pl.ds(s * RPT, RPT)])
        pltpu.sync_copy(zbuf, tab.at[pl.ds(s * RPT, RPT)])
        plsc.subcore_barrier()


def _layer_body(h_hbm, srcw, dstw, out, acc, sidx, didx, gbuf, zbuf, sem0, sem1):
    c = lax.axis_index("c")
    s = lax.axis_index("s")
    w = c * 16 + s
    _zero_rows(zbuf, RPT, NH)
    pltpu.sync_copy(zbuf, acc.at[pl.ds(s * RPT, RPT)])
    plsc.subcore_barrier()
    for i in range(NE):
        pltpu.sync_copy(srcw.at[pl.ds((i * NW + w) * NC, NC)], sidx)
        pltpu.sync_copy(dstw.at[pl.ds((i * NW + w) * NC, NC)], didx)
        pltpu.async_copy(h_hbm.at[sidx.at[0]], gbuf.at[0], sem0)

        @pl.loop(0, NC, step=2)
        def _(ch):
            d1 = pltpu.async_copy(h_hbm.at[sidx.at[ch + 1]], gbuf.at[1], sem1)
            pltpu.make_async_copy(h_hbm.at[sidx.at[ch]], gbuf.at[0], sem0).wait()
            pltpu.sync_copy(gbuf.at[0], acc.at[didx.at[ch]], add=True)

            @pl.when(ch + 2 < NC)
            def _():
                pltpu.async_copy(h_hbm.at[sidx.at[ch + 2]], gbuf.at[0], sem0)

            d1.wait()
            pltpu.sync_copy(gbuf.at[1], acc.at[didx.at[ch + 1]], add=True)

        plsc.subcore_barrier()
        pltpu.sync_copy(acc.at[pl.ds(s * RPT, RPT)],
                        out.at[c, i, pl.ds(s * RPT, RPT)])
        pltpu.sync_copy(zbuf, acc.at[pl.ds(s * RPT, RPT)])
        plsc.subcore_barrier()


_sc_params = pltpu.CompilerParams(use_tc_tiling_on_sc=False)

_pass0 = functools.partial(
    pl.kernel,
    _pass0_body,
    out_type=jax.ShapeDtypeStruct((2, NE, NPAD, 16), jnp.float32),
    mesh=_mesh,
    compiler_params=_sc_params,
    scratch_types=[
        pltpu.VMEM_SHARED((NPAD, 16), jnp.float32),
        pltpu.VMEM((NC, CK), jnp.int32),
        pltpu.VMEM((2, CK, 16), jnp.float32),
        pltpu.VMEM((RPT, 16), jnp.float32),
        pltpu.SemaphoreType.DMA,
        pltpu.SemaphoreType.DMA,
    ],
)()

_layer_agg = functools.partial(
    pl.kernel,
    _layer_body,
    out_type=jax.ShapeDtypeStruct((2, NE, NPAD, NH), jnp.float32),
    mesh=_mesh,
    compiler_params=_sc_params,
    scratch_types=[
        pltpu.VMEM_SHARED((NPAD, NH), jnp.float32),
        pltpu.VMEM((NC, CK), jnp.int32),
        pltpu.VMEM((NC, CK), jnp.int32),
        pltpu.VMEM((2, CK, NH), jnp.float32),
        pltpu.VMEM((RPT, NH), jnp.float32),
        pltpu.SemaphoreType.DMA,
        pltpu.SemaphoreType.DMA,
    ],
)()


def kernel(op_feats, tensor_feats, edge_index, W_op, b_op, W_edge, b_edge,
           W_gc, b_gc, W_fin, b_fin):
    f32 = jnp.float32

    src = edge_index[:, 0, :].reshape(NE, NW, EPW)
    dst = edge_index[:, 1, :].reshape(NE, NW, EPW)
    pad_src = jnp.broadcast_to((jnp.arange(PADN, dtype=jnp.int32) * 37) % N,
                               (NE, NW, PADN))
    pad_dst = jnp.broadcast_to(N + (jnp.arange(PADN, dtype=jnp.int32) % (NPAD - N)),
                               (NE, NW, PADN))
    srcW = jnp.concatenate([src, pad_src], axis=2).reshape(IDXR, CK)
    dstW = jnp.concatenate([dst, pad_dst], axis=2).reshape(IDXR, CK)

    opf_pad = jnp.concatenate(
        [op_feats, jnp.zeros((NPAD - N, D_OP), f32)], axis=0)
    h = pl.pallas_call(
        _h0_body,
        out_shape=jax.ShapeDtypeStruct((NPAD, NH), f32),
    )(opf_pad, W_op, b_op)

    tf8 = tensor_feats.reshape(NW, EPW8, 8 * D_T)
    tf8 = jnp.concatenate(
        [tf8, jnp.zeros((NW, SLOTS8 - EPW8, 8 * D_T), f32)],
        axis=1).reshape(NW * SLOTS8, 8 * D_T)
    W8 = jnp.zeros((NE, 128, 128), f32)
    b8 = jnp.zeros((NE, 128), f32)
    for j in range(8):
        W8 = W8.at[:, j * 16:j * 16 + D_T, j * 16:j * 16 + EH].set(W_edge)
        b8 = b8.at[:, j * 16:j * 16 + EH].set(b_edge)
        b8 = b8.at[:, j * 16 + EH].set(1.0)
    y8 = pl.pallas_call(
        _efsp_body,
        grid=(NE, NW),
        in_specs=[
            pl.BlockSpec((SLOTS8, 128), lambda i, w: (w, 0)),
            pl.BlockSpec((1, 128, 128), lambda i, w: (i, 0, 0)),
            pl.BlockSpec((1, 1, 128), lambda i, w: (i, 0, 0)),
        ],
        out_specs=pl.BlockSpec((1, 1, SLOTS8, 128), lambda i, w: (i, w, 0, 0)),
        out_shape=jax.ShapeDtypeStruct((NE, NW, SLOTS8, 128), f32),
    )(tf8, W8, b8[:, None, :])
    efsp = y8.reshape(NE, NW, SLOTS, 16)

    p0 = _pass0(efsp, dstW)
    S = p0[0] + p0[1]
    cnt = S[..., EH]
    invc = 1.0 / jnp.maximum(cnt, 1.0)
    mask = (cnt > 0).astype(f32)
    M_e = S[..., :EH] * invc[..., None]

    W1 = W_gc[:, :, :NH, :]
    W2 = W_gc[:, :, NH:, :]

    BN = 2528
    for l in range(L):
        p = _layer_agg(h, srcW, dstW)
        h = pl.pallas_call(
            functools.partial(_combine_body, l == L - 1),
            grid=(NPAD // BN,),
            in_specs=[
                pl.BlockSpec((BN, NH), lambda r: (r, 0)),
                pl.BlockSpec((2, NE, BN, NH), lambda r: (0, 0, r, 0)),
                pl.BlockSpec((BN, NE), lambda r: (r, 0)),
                pl.BlockSpec((BN, NE), lambda r: (r, 0)),
                pl.BlockSpec((NE, BN, EH), lambda r: (0, r, 0)),
                pl.BlockSpec((NE, NH, NH), lambda r: (0, 0, 0)),
                pl.BlockSpec((NE, EH, NH), lambda r: (0, 0, 0)),
                pl.BlockSpec((NE, NH), lambda r: (0, 0)),
            ],
            out_specs=pl.BlockSpec((BN, NH), lambda r: (r, 0)),
            out_shape=jax.ShapeDtypeStruct((NPAD, NH), f32),
        )(h, p, invc.T, mask.T, M_e, W1[l], W2[l], b_gc[l])

    out = pl.pallas_call(
        _final_body,
        out_shape=jax.ShapeDtypeStruct((NPAD, 1), f32),
    )(h, W_fin, b_fin)
    return out[:N, 0]


# speedup vs baseline: 1.3507x; 1.3507x over previous
import functools

import jax
import jax.numpy as jnp
import numpy as np
from jax import lax
from jax.experimental import pallas as pl
from jax.experimental.pallas import tpu as pltpu
from jax.experimental.pallas import tpu_sc as plsc

N = 10000
E = 320000
NE = 5
D_OP = 128
D_T = 16
NH = 64
EH = 8
L = 6

NPAD = 10112
NW = 32
EPW = E // NW
CK = 128
NC = 80
SLOTS = NC * CK
PADN = SLOTS - EPW
RPT = NPAD // 16
EPW8 = EPW // 8
SLOTS8 = SLOTS // 8
IDXR = NE * NW * NC

_mesh = plsc.VectorSubcoreMesh(core_axis_name="c", subcore_axis_name="s")

_SEL_IN = np.zeros((8, 128, D_T), np.float32)
_SEL_OUT = np.zeros((8, 128, EH), np.float32)
_CNT_COL = np.zeros((128,), np.float32)
for _j in range(8):
    for _k in range(D_T):
        _SEL_IN[_j, 16 * _j + _k, _k] = 1.0
    for _c in range(EH):
        _SEL_OUT[_j, 16 * _j + _c, _c] = 1.0
    _CNT_COL[16 * _j + EH] = 1.0


def _elu(x):
    return jnp.where(x > 0, x, jnp.exp(jnp.minimum(x, 0.0)) - 1.0)


def _h0_body(x_ref, w_ref, b_ref, o_ref):
    o_ref[...] = _elu(
        jnp.dot(x_ref[...], w_ref[...], preferred_element_type=jnp.float32)
        + b_ref[...][None, :])


def _efsp_body(tf_ref, w_ref, b_ref, o_ref):
    o_ref[0, 0] = _elu(
        jnp.dot(tf_ref[...], w_ref[0], preferred_element_type=jnp.float32)
        + b_ref[0, 0][None, :])


def _combine_body(last, h_ref, p_ref, invc_ref, mask_ref, me_ref,
                  w1_ref, w2_ref, b_ref, o_ref):
    h = h_ref[...]
    acc = jnp.zeros_like(h)
    for i in range(NE):
        t = (p_ref[0, i] + p_ref[1, i]) * invc_ref[:, i][:, None]
        acc = acc + jnp.dot(t, w1_ref[i], preferred_element_type=jnp.float32)
        acc = acc + jnp.dot(me_ref[i], w2_ref[i], preferred_element_type=jnp.float32)
        acc = acc + mask_ref[:, i][:, None] * b_ref[i][None, :]
    r = h + acc * (1.0 / NE)
    o_ref[...] = r if last else _elu(r)


def _final_body(h_ref, w_ref, b_ref, o_ref):
    o_ref[...] = jnp.dot(h_ref[...], w_ref[...],
                         preferred_element_type=jnp.float32) + b_ref[0]


def _zero_rows(buf, rows, width):
    z = jnp.zeros((16,), jnp.float32)

    @pl.loop(0, rows)
    def _(r):
        for k in range(width // 16):
            buf[r, pl.ds(k * 16, 16)] = z


def _unpack16(src, dst):
    for e in range(CK):
        dst[e, pl.ds(0, 16)] = src[e // 8, pl.ds(16 * (e % 8), 16)]


def _pass0_body(efsp, dstw, out, tab, idx, ebuf, sbuf, zbuf, sem0, sem1):
    c = lax.axis_index("c")
    s = lax.axis_index("s")
    w = c * 16 + s
    _zero_rows(zbuf, RPT, 16)
    pltpu.sync_copy(zbuf, tab.at[pl.ds(s * RPT, RPT)])
    plsc.subcore_barrier()
    for i in range(NE):
        pltpu.sync_copy(dstw.at[pl.ds((i * NW + w) * NC, NC)], idx)
        pltpu.async_copy(efsp.at[i, w, pl.ds(0, 16)], ebuf.at[0], sem0)

        @pl.loop(0, NC, step=2)
        def _(ch):
            d1 = pltpu.async_copy(
                efsp.at[i, w, pl.ds((ch + 1) * 16, 16)], ebuf.at[1], sem1)
            pltpu.make_async_copy(
                efsp.at[i, w, pl.ds(ch * 16, 16)], ebuf.at[0], sem0).wait()
            _unpack16(ebuf.at[0], sbuf.at[0])
            pltpu.sync_copy(sbuf.at[0], tab.at[idx.at[ch]], add=True)

            @pl.when(ch + 2 < NC)
            def _():
                pltpu.async_copy(
                    efsp.at[i, w, pl.ds((ch + 2) * 16, 16)], ebuf.at[0], sem0)

            d1.wait()
            _unpack16(ebuf.at[1], sbuf.at[1])
            pltpu.sync_copy(sbuf.at[1], tab.at[idx.at[ch + 1]], add=True)

        plsc.subcore_barrier()
        pltpu.sync_copy(tab.at[pl.ds(s * RPT, RPT)],
                        out.at[c, i, pl.ds(s * RPT, RPT)])
        pltpu.sync_copy(zbuf, tab.at[pl.ds(s * RPT, RPT)])
        plsc.subcore_barrier()


def _layer_body(h_hbm, srcw, dstw, out, acc, sidx, didx, gbuf, zbuf, sem0, sem1):
    c = lax.axis_index("c")
    s = lax.axis_index("s")
    w = c * 16 + s
    _zero_rows(zbuf, RPT, NH)
    pltpu.sync_copy(zbuf, acc.at[pl.ds(s * RPT, RPT)])
    plsc.subcore_barrier()
    for i in range(NE):
        pltpu.sync_copy(srcw.at[pl.ds((i * NW + w) * NC, NC)], sidx)
        pltpu.sync_copy(dstw.at[pl.ds((i * NW + w) * NC, NC)], didx)
        pltpu.async_copy(h_hbm.at[sidx.at[0]], gbuf.at[0], sem0)

        @pl.loop(0, NC, step=2)
        def _(ch):
            d1 = pltpu.async_copy(h_hbm.at[sidx.at[ch + 1]], gbuf.at[1], sem1)
            pltpu.make_async_copy(h_hbm.at[sidx.at[ch]], gbuf.at[0], sem0).wait()
            pltpu.sync_copy(gbuf.at[0], acc.at[didx.at[ch]], add=True)

            @pl.when(ch + 2 < NC)
            def _():
                pltpu.async_copy(h_hbm.at[sidx.at[ch + 2]], gbuf.at[0], sem0)

            d1.wait()
            pltpu.sync_copy(gbuf.at[1], acc.at[didx.at[ch + 1]], add=True)

        plsc.subcore_barrier()
        pltpu.sync_copy(acc.at[pl.ds(s * RPT, RPT)],
                        out.at[c, i, pl.ds(s * RPT, RPT)])
        pltpu.sync_copy(zbuf, acc.at[pl.ds(s * RPT, RPT)])
        plsc.subcore_barrier()


_sc_params = pltpu.CompilerParams(use_tc_tiling_on_sc=False)

_pass0 = functools.partial(
    pl.kernel,
    _pass0_body,
    out_type=jax.ShapeDtypeStruct((2, NE, NPAD, 16), jnp.float32),
    mesh=_mesh,
    compiler_params=_sc_params,
    scratch_types=[
        pltpu.VMEM_SHARED((NPAD, 16), jnp.float32),
        pltpu.VMEM((NC, CK), jnp.int32),
        pltpu.VMEM((2, 16, 128), jnp.float32),
        pltpu.VMEM((2, CK, 16), jnp.float32),
        pltpu.VMEM((RPT, 16), jnp.float32),
        pltpu.SemaphoreType.DMA,
        pltpu.SemaphoreType.DMA,
    ],
)()

_layer_agg = functools.partial(
    pl.kernel,
    _layer_body,
    out_type=jax.ShapeDtypeStruct((2, NE, NPAD, NH), jnp.float32),
    mesh=_mesh,
    compiler_params=_sc_params,
    scratch_types=[
        pltpu.VMEM_SHARED((NPAD, NH), jnp.float32),
        pltpu.VMEM((NC, CK), jnp.int32),
        pltpu.VMEM((NC, CK), jnp.int32),
        pltpu.VMEM((2, CK, NH), jnp.float32),
        pltpu.VMEM((RPT, NH), jnp.float32),
        pltpu.SemaphoreType.DMA,
        pltpu.SemaphoreType.DMA,
    ],
)()


def kernel(op_feats, tensor_feats, edge_index, W_op, b_op, W_edge, b_edge,
           W_gc, b_gc, W_fin, b_fin):
    f32 = jnp.float32

    src = edge_index[:, 0, :].reshape(NE, NW, EPW)
    dst = edge_index[:, 1, :].reshape(NE, NW, EPW)
    pad_src = jnp.broadcast_to((jnp.arange(PADN, dtype=jnp.int32) * 37) % N,
                               (NE, NW, PADN))
    pad_dst = jnp.broadcast_to(N + (jnp.arange(PADN, dtype=jnp.int32) % (NPAD - N)),
                               (NE, NW, PADN))
    srcW = jnp.concatenate([src, pad_src], axis=2).reshape(IDXR, CK)
    dstW = jnp.concatenate([dst, pad_dst], axis=2).reshape(IDXR, CK)

    opf_pad = jnp.concatenate(
        [op_feats, jnp.zeros((NPAD - N, D_OP), f32)], axis=0)
    h = pl.pallas_call(
        _h0_body,
        out_shape=jax.ShapeDtypeStruct((NPAD, NH), f32),
    )(opf_pad, W_op, b_op)

    tf8 = tensor_feats.reshape(NW, EPW8, 8 * D_T)
    tf8 = jnp.concatenate(
        [tf8, jnp.zeros((NW, SLOTS8 - EPW8, 8 * D_T), f32)],
        axis=1).reshape(NW * SLOTS8, 8 * D_T)
    W8 = jnp.einsum('jak,ikc,jbc->iab', _SEL_IN, W_edge, _SEL_OUT)
    b8 = jnp.einsum('jbc,ic->ib', _SEL_OUT, b_edge) + _CNT_COL[None, :]
    y8 = pl.pallas_call(
        _efsp_body,
        grid=(NE, NW),
        in_specs=[
            pl.BlockSpec((SLOTS8, 128), lambda i, w: (w, 0)),
            pl.BlockSpec((1, 128, 128), lambda i, w: (i, 0, 0)),
            pl.BlockSpec((1, 1, 128), lambda i, w: (i, 0, 0)),
        ],
        out_specs=pl.BlockSpec((1, 1, SLOTS8, 128), lambda i, w: (i, w, 0, 0)),
        out_shape=jax.ShapeDtypeStruct((NE, NW, SLOTS8, 128), f32),
    )(tf8, W8, b8[:, None, :])

    p0 = _pass0(y8, dstW)
    S = p0[0] + p0[1]
    cnt = S[..., EH]
    invc = 1.0 / jnp.maximum(cnt, 1.0)
    mask = (cnt > 0).astype(f32)
    M_e = S[..., :EH] * invc[..., None]

    W1 = W_gc[:, :, :NH, :]
    W2 = W_gc[:, :, NH:, :]

    BN = 2528
    for l in range(L):
        p = _layer_agg(h, srcW, dstW)
        h = pl.pallas_call(
            functools.partial(_combine_body, l == L - 1),
            grid=(NPAD // BN,),
            in_specs=[
                pl.BlockSpec((BN, NH), lambda r: (r, 0)),
                pl.BlockSpec((2, NE, BN, NH), lambda r: (0, 0, r, 0)),
                pl.BlockSpec((BN, NE), lambda r: (r, 0)),
                pl.BlockSpec((BN, NE), lambda r: (r, 0)),
                pl.BlockSpec((NE, BN, EH), lambda r: (0, r, 0)),
                pl.BlockSpec((NE, NH, NH), lambda r: (0, 0, 0)),
                pl.BlockSpec((NE, EH, NH), lambda r: (0, 0, 0)),
                pl.BlockSpec((NE, NH), lambda r: (0, 0)),
            ],
            out_specs=pl.BlockSpec((BN, NH), lambda r: (r, 0)),
            out_shape=jax.ShapeDtypeStruct((NPAD, NH), f32),
        )(h, p, invc.T, mask.T, M_e, W1[l], W2[l], b_gc[l])

    out = pl.pallas_call(
        _final_body,
        out_shape=jax.ShapeDtypeStruct((NPAD, 1), f32),
    )(h, W_fin, b_fin)
    return out[:N, 0]


# speedup vs baseline: 1.3663x; 1.0115x over previous
import functools

import jax
import jax.numpy as jnp
import numpy as np
from jax import lax
from jax.experimental import pallas as pl
from jax.experimental.pallas import tpu as pltpu
from jax.experimental.pallas import tpu_sc as plsc

N = 10000
E = 320000
NE = 5
D_OP = 128
D_T = 16
NH = 64
EH = 8
L = 6

NPAD = 10112
NW = 32
EPW = E // NW
CK = 128
NC = 80
SLOTS = NC * CK
PADN = SLOTS - EPW
RPT = NPAD // 16
EPW8 = EPW // 8
SLOTS8 = SLOTS // 8
IDXR = NE * NW * NC

_mesh = plsc.VectorSubcoreMesh(core_axis_name="c", subcore_axis_name="s")

_SEL_IN = np.zeros((8, 128, D_T), np.float32)
_SEL_OUT = np.zeros((8, 128, EH), np.float32)
_CNT_COL = np.zeros((128,), np.float32)
for _j in range(8):
    for _k in range(D_T):
        _SEL_IN[_j, 16 * _j + _k, _k] = 1.0
    for _c in range(EH):
        _SEL_OUT[_j, 16 * _j + _c, _c] = 1.0
    _CNT_COL[16 * _j + EH] = 1.0


def _elu(x):
    return jnp.where(x > 0, x, jnp.exp(jnp.minimum(x, 0.0)) - 1.0)


def _h0_body(x_ref, w_ref, b_ref, o_ref):
    o_ref[...] = _elu(
        jnp.dot(x_ref[...], w_ref[...], preferred_element_type=jnp.float32)
        + b_ref[...][None, :])


def _efsp_body(tf_ref, w_ref, b_ref, o_ref):
    y = _elu(
        jnp.dot(tf_ref[...], w_ref[0], preferred_element_type=jnp.float32)
        + b_ref[0, 0][None, :])
    zpad = jnp.zeros((SLOTS8 - EPW8, 128), jnp.float32)
    for k in range(4):
        o_ref[0, k] = jnp.concatenate(
            [y[k * EPW8:(k + 1) * EPW8], zpad], axis=0)


def _combine_body(last, h_ref, p_ref, invc_ref, mask_ref, me_ref,
                  w1_ref, w2_ref, b_ref, o_ref):
    h = h_ref[...]
    acc = jnp.zeros_like(h)
    for i in range(NE):
        t = (p_ref[0, i] + p_ref[1, i]) * invc_ref[:, i][:, None]
        acc = acc + jnp.dot(t, w1_ref[i], preferred_element_type=jnp.float32)
        acc = acc + jnp.dot(me_ref[i], w2_ref[i], preferred_element_type=jnp.float32)
        acc = acc + mask_ref[:, i][:, None] * b_ref[i][None, :]
    r = h + acc * (1.0 / NE)
    o_ref[...] = r if last else _elu(r)


def _final_body(h_ref, w_ref, b_ref, o_ref):
    o_ref[...] = jnp.dot(h_ref[...], w_ref[...],
                         preferred_element_type=jnp.float32) + b_ref[0]


def _zero_rows(buf, rows, width):
    z = jnp.zeros((16,), jnp.float32)

    @pl.loop(0, rows)
    def _(r):
        for k in range(width // 16):
            buf[r, pl.ds(k * 16, 16)] = z


def _unpack16(src, dst):
    for e in range(CK):
        dst[e, pl.ds(0, 16)] = src[e // 8, pl.ds(16 * (e % 8), 16)]


def _unpack_idx(pidx, sidx, didx):
    @pl.loop(0, NC)
    def _(r):
        for k in range(CK // 16):
            v = pidx[r, pl.ds(k * 16, 16)]
            if sidx is not None:
                sidx[r, pl.ds(k * 16, 16)] = v & 0xFFFF
            didx[r, pl.ds(k * 16, 16)] = v >> 16


def _pass0_body(efsp, pkw, out, tab, pidx, idx, ebuf, sbuf, zbuf, sem0, sem1):
    c = lax.axis_index("c")
    s = lax.axis_index("s")
    w = c * 16 + s
    _zero_rows(zbuf, RPT, 16)
    pltpu.sync_copy(zbuf, tab.at[pl.ds(s * RPT, RPT)])
    plsc.subcore_barrier()
    for i in range(NE):
        pltpu.sync_copy(pkw.at[pl.ds((i * NW + w) * NC, NC)], pidx)
        _unpack_idx(pidx, None, idx)
        pltpu.async_copy(efsp.at[i, w, pl.ds(0, 16)], ebuf.at[0], sem0)

        @pl.loop(0, NC, step=2)
        def _(ch):
            d1 = pltpu.async_copy(
                efsp.at[i, w, pl.ds((ch + 1) * 16, 16)], ebuf.at[1], sem1)
            pltpu.make_async_copy(
                efsp.at[i, w, pl.ds(ch * 16, 16)], ebuf.at[0], sem0).wait()
            _unpack16(ebuf.at[0], sbuf.at[0])
            pltpu.sync_copy(sbuf.at[0], tab.at[idx.at[ch]], add=True)

            @pl.when(ch + 2 < NC)
            def _():
                pltpu.async_copy(
                    efsp.at[i, w, pl.ds((ch + 2) * 16, 16)], ebuf.at[0], sem0)

            d1.wait()
            _unpack16(ebuf.at[1], sbuf.at[1])
            pltpu.sync_copy(sbuf.at[1], tab.at[idx.at[ch + 1]], add=True)

        plsc.subcore_barrier()
        pltpu.sync_copy(tab.at[pl.ds(s * RPT, RPT)],
                        out.at[c, i, pl.ds(s * RPT, RPT)])
        pltpu.sync_copy(zbuf, tab.at[pl.ds(s * RPT, RPT)])
        plsc.subcore_barrier()


def _layer_body(h_hbm, pkw, out, acc, pidx, sidx, didx, gbuf, zbuf, sem0, sem1):
    c = lax.axis_index("c")
    s = lax.axis_index("s")
    w = c * 16 + s
    _zero_rows(zbuf, RPT, NH)
    pltpu.sync_copy(zbuf, acc.at[pl.ds(s * RPT, RPT)])
    plsc.subcore_barrier()
    for i in range(NE):
        pltpu.sync_copy(pkw.at[pl.ds((i * NW + w) * NC, NC)], pidx)
        _unpack_idx(pidx, sidx, didx)
        pltpu.async_copy(h_hbm.at[sidx.at[0]], gbuf.at[0], sem0)

        @pl.loop(0, NC, step=2)
        def _(ch):
            d1 = pltpu.async_copy(h_hbm.at[sidx.at[ch + 1]], gbuf.at[1], sem1)
            pltpu.make_async_copy(h_hbm.at[sidx.at[ch]], gbuf.at[0], sem0).wait()
            pltpu.sync_copy(gbuf.at[0], acc.at[didx.at[ch]], add=True)

            @pl.when(ch + 2 < NC)
            def _():
                pltpu.async_copy(h_hbm.at[sidx.at[ch + 2]], gbuf.at[0], sem0)

            d1.wait()
            pltpu.sync_copy(gbuf.at[1], acc.at[didx.at[ch + 1]], add=True)

        plsc.subcore_barrier()
        pltpu.sync_copy(acc.at[pl.ds(s * RPT, RPT)],
                        out.at[c, i, pl.ds(s * RPT, RPT)])
        pltpu.sync_copy(zbuf, acc.at[pl.ds(s * RPT, RPT)])
        plsc.subcore_barrier()


_sc_params = pltpu.CompilerParams(use_tc_tiling_on_sc=False)

_pass0 = functools.partial(
    pl.kernel,
    _pass0_body,
    out_type=jax.ShapeDtypeStruct((2, NE, NPAD, 16), jnp.float32),
    mesh=_mesh,
    compiler_params=_sc_params,
    scratch_types=[
        pltpu.VMEM_SHARED((NPAD, 16), jnp.float32),
        pltpu.VMEM((NC, CK), jnp.int32),
        pltpu.VMEM((NC, CK), jnp.int32),
        pltpu.VMEM((2, 16, 128), jnp.float32),
        pltpu.VMEM((2, CK, 16), jnp.float32),
        pltpu.VMEM((RPT, 16), jnp.float32),
        pltpu.SemaphoreType.DMA,
        pltpu.SemaphoreType.DMA,
    ],
)()

_layer_agg = functools.partial(
    pl.kernel,
    _layer_body,
    out_type=jax.ShapeDtypeStruct((2, NE, NPAD, NH), jnp.float32),
    mesh=_mesh,
    compiler_params=_sc_params,
    scratch_types=[
        pltpu.VMEM_SHARED((NPAD, NH), jnp.float32),
        pltpu.VMEM((NC, CK), jnp.int32),
        pltpu.VMEM((NC, CK), jnp.int32),
        pltpu.VMEM((NC, CK), jnp.int32),
        pltpu.VMEM((2, CK, NH), jnp.float32),
        pltpu.VMEM((RPT, NH), jnp.float32),
        pltpu.SemaphoreType.DMA,
        pltpu.SemaphoreType.DMA,
    ],
)()


def kernel(op_feats, tensor_feats, edge_index, W_op, b_op, W_edge, b_edge,
           W_gc, b_gc, W_fin, b_fin):
    f32 = jnp.float32

    src = edge_index[:, 0, :].reshape(NE, NW, EPW)
    dst = edge_index[:, 1, :].reshape(NE, NW, EPW)
    pad_src = jnp.broadcast_to((jnp.arange(PADN, dtype=jnp.int32) * 37) % N,
                               (NE, NW, PADN))
    pad_dst = jnp.broadcast_to(N + (jnp.arange(PADN, dtype=jnp.int32) % (NPAD - N)),
                               (NE, NW, PADN))
    pkW = jnp.concatenate([src + dst * 65536, pad_src + pad_dst * 65536],
                          axis=2).reshape(IDXR, CK)

    opf_pad = jnp.concatenate(
        [op_feats, jnp.zeros((NPAD - N, D_OP), f32)], axis=0)
    h = pl.pallas_call(
        _h0_body,
        out_shape=jax.ShapeDtypeStruct((NPAD, NH), f32),
    )(opf_pad, W_op, b_op)

    tf8 = tensor_feats.reshape(E // 8, 8 * D_T)
    W8 = jnp.einsum('jak,ikc,jbc->iab', _SEL_IN, W_edge, _SEL_OUT)
    b8 = jnp.einsum('jbc,ic->ib', _SEL_OUT, b_edge) + _CNT_COL[None, :]
    y8 = pl.pallas_call(
        _efsp_body,
        grid=(NE, NW // 4),
        in_specs=[
            pl.BlockSpec((4 * EPW8, 128), lambda i, w: (w, 0)),
            pl.BlockSpec((1, 128, 128), lambda i, w: (i, 0, 0)),
            pl.BlockSpec((1, 1, 128), lambda i, w: (i, 0, 0)),
        ],
        out_specs=pl.BlockSpec((1, 4, SLOTS8, 128), lambda i, w: (i, w, 0, 0)),
        out_shape=jax.ShapeDtypeStruct((NE, NW, SLOTS8, 128), f32),
    )(tf8, W8, b8[:, None, :])

    p0 = _pass0(y8, pkW)
    S = p0[0] + p0[1]
    cnt = S[..., EH]
    invc = 1.0 / jnp.maximum(cnt, 1.0)
    mask = (cnt > 0).astype(f32)
    M_e = S[..., :EH] * invc[..., None]

    W1 = W_gc[:, :, :NH, :]
    W2 = W_gc[:, :, NH:, :]

    BN = 2528
    for l in range(L):
        p = _layer_agg(h, pkW)
        h = pl.pallas_call(
            functools.partial(_combine_body, l == L - 1),
            grid=(NPAD // BN,),
            in_specs=[
                pl.BlockSpec((BN, NH), lambda r: (r, 0)),
                pl.BlockSpec((2, NE, BN, NH), lambda r: (0, 0, r, 0)),
                pl.BlockSpec((BN, NE), lambda r: (r, 0)),
                pl.BlockSpec((BN, NE), lambda r: (r, 0)),
                pl.BlockSpec((NE, BN, EH), lambda r: (0, r, 0)),
                pl.BlockSpec((NE, NH, NH), lambda r: (0, 0, 0)),
                pl.BlockSpec((NE, EH, NH), lambda r: (0, 0, 0)),
                pl.BlockSpec((NE, NH), lambda r: (0, 0)),
            ],
            out_specs=pl.BlockSpec((BN, NH), lambda r: (r, 0)),
            out_shape=jax.ShapeDtypeStruct((NPAD, NH), f32),
        )(h, p, invc.T, mask.T, M_e, W1[l], W2[l], b_gc[l])

    out = pl.pallas_call(
        _final_body,
        out_shape=jax.ShapeDtypeStruct((NPAD, 1), f32),
    )(h, W_fin, b_fin)
    return out[:N, 0]


# speedup vs baseline: 1.4726x; 1.0777x over previous
import functools

import jax
import jax.numpy as jnp
import numpy as np
from jax import lax
from jax.experimental import pallas as pl
from jax.experimental.pallas import tpu as pltpu
from jax.experimental.pallas import tpu_sc as plsc

N = 10000
E = 320000
NE = 5
D_OP = 128
D_T = 16
NH = 64
EH = 8
L = 6

NPAD = 10112
NW = 32
EPW = E // NW
CK = 128
NC = 80
SLOTS = NC * CK
PADN = SLOTS - EPW
RPT = NPAD // 16
EPW8 = EPW // 8
SLOTS8 = SLOTS // 8
IDXR = NE * NW * NC

_mesh = plsc.VectorSubcoreMesh(core_axis_name="c", subcore_axis_name="s")

_SEL_IN = np.zeros((8, 128, D_T), np.float32)
_SEL_OUT = np.zeros((8, 128, EH), np.float32)
_CNT_COL = np.zeros((128,), np.float32)
for _j in range(8):
    for _k in range(D_T):
        _SEL_IN[_j, 16 * _j + _k, _k] = 1.0
    for _c in range(EH):
        _SEL_OUT[_j, 16 * _j + _c, _c] = 1.0
    _CNT_COL[16 * _j + EH] = 1.0


def _elu(x):
    return jnp.where(x > 0, x, jnp.exp(jnp.minimum(x, 0.0)) - 1.0)


def _h0_body(x_ref, w_ref, b_ref, o_ref):
    o_ref[...] = _elu(
        jnp.dot(x_ref[...], w_ref[...], preferred_element_type=jnp.float32)
        + b_ref[...][None, :])


def _efsp_body(tf_ref, w_ref, b_ref, o_ref):
    y = _elu(
        jnp.dot(tf_ref[...], w_ref[0], preferred_element_type=jnp.float32)
        + b_ref[0, 0][None, :])
    zpad = jnp.zeros((SLOTS8 - EPW8, 128), jnp.float32)
    for k in range(4):
        o_ref[0, k] = jnp.concatenate(
            [y[k * EPW8:(k + 1) * EPW8], zpad], axis=0)


def _combine_body(last, h_ref, p_ref, invc_ref, mask_ref, me_ref,
                  w1_ref, w2_ref, b_ref, o_ref):
    h = h_ref[...]
    acc = jnp.zeros_like(h)
    for i in range(NE):
        t = (p_ref[0, i] + p_ref[1, i]) * invc_ref[:, i][:, None]
        acc = acc + jnp.dot(t, w1_ref[i], preferred_element_type=jnp.float32)
        acc = acc + jnp.dot(me_ref[i], w2_ref[i], preferred_element_type=jnp.float32)
        acc = acc + mask_ref[:, i][:, None] * b_ref[i][None, :]
    r = h + acc * (1.0 / NE)
    o_ref[...] = r if last else _elu(r)


def _final_body(h_ref, w_ref, b_ref, o_ref):
    o_ref[...] = jnp.dot(h_ref[...], w_ref[...],
                         preferred_element_type=jnp.float32) + b_ref[0]


def _zero_rows(buf, rows, width):
    z = jnp.zeros((16,), jnp.float32)

    @pl.loop(0, rows)
    def _(r):
        for k in range(width // 16):
            buf[r, pl.ds(k * 16, 16)] = z


def _unpack16(src, dst):
    for e in range(CK):
        dst[e, pl.ds(0, 16)] = src[e // 8, pl.ds(16 * (e % 8), 16)]


def _unpack_idx(pidx, sidx, didx):
    @pl.loop(0, NC)
    def _(r):
        for k in range(CK // 16):
            v = pidx[r, pl.ds(k * 16, 16)]
            if sidx is not None:
                sidx[r, pl.ds(k * 16, 16)] = v & 0xFFFF
            didx[r, pl.ds(k * 16, 16)] = v >> 16


def _pass0_body(efsp, pkw, out, tab, pidx, idx, ebuf, sbuf, zbuf, sem0, sem1):
    c = lax.axis_index("c")
    s = lax.axis_index("s")
    w = c * 16 + s
    _zero_rows(zbuf, RPT, 16)
    pltpu.sync_copy(zbuf, tab.at[pl.ds(s * RPT, RPT)])
    plsc.subcore_barrier()
    for i in range(NE):
        pltpu.sync_copy(pkw.at[pl.ds((i * NW + w) * NC, NC)], pidx)
        _unpack_idx(pidx, None, idx)
        pltpu.async_copy(efsp.at[i, w, pl.ds(0, 16)], ebuf.at[0], sem0)

        @pl.loop(0, NC, step=2)
        def _(ch):
            d1 = pltpu.async_copy(
                efsp.at[i, w, pl.ds((ch + 1) * 16, 16)], ebuf.at[1], sem1)
            pltpu.make_async_copy(
                efsp.at[i, w, pl.ds(ch * 16, 16)], ebuf.at[0], sem0).wait()
            _unpack16(ebuf.at[0], sbuf.at[0])
            pltpu.sync_copy(sbuf.at[0], tab.at[idx.at[ch]], add=True)

            @pl.when(ch + 2 < NC)
            def _():
                pltpu.async_copy(
                    efsp.at[i, w, pl.ds((ch + 2) * 16, 16)], ebuf.at[0], sem0)

            d1.wait()
            _unpack16(ebuf.at[1], sbuf.at[1])
            pltpu.sync_copy(sbuf.at[1], tab.at[idx.at[ch + 1]], add=True)

        plsc.subcore_barrier()
        pltpu.sync_copy(tab.at[pl.ds(s * RPT, RPT)],
                        out.at[c, i, pl.ds(s * RPT, RPT)])
        pltpu.sync_copy(zbuf, tab.at[pl.ds(s * RPT, RPT)])
        plsc.subcore_barrier()


def _layer_body(h_hbm, pkw, out, acc, pidx, sidx, didx, gbuf, zbuf,
                sg0, sg1, sg2, sg3, ss0, ss1, ss2, ss3):
    c = lax.axis_index("c")
    s = lax.axis_index("s")
    w = c * 16 + s
    sg = (sg0, sg1, sg2, sg3)
    ss = (ss0, ss1, ss2, ss3)
    _zero_rows(zbuf, RPT // 8, NH)

    def _zero_acc():
        for r8 in range(8):
            pltpu.sync_copy(
                zbuf, acc.at[pl.ds(s * RPT + r8 * (RPT // 8), RPT // 8)])

    _zero_acc()
    plsc.subcore_barrier()
    for i in range(NE):
        pltpu.sync_copy(pkw.at[pl.ds((i * NW + w) * NC, NC)], pidx)
        _unpack_idx(pidx, sidx, didx)
        for k in range(4):
            pltpu.async_copy(h_hbm.at[sidx.at[k]], gbuf.at[k], sg[k])

        @pl.loop(0, NC, step=4)
        def _(ch):
            for k in range(4):
                pltpu.make_async_copy(
                    h_hbm.at[sidx.at[ch + k]], gbuf.at[k], sg[k]).wait()
                pltpu.async_copy(
                    gbuf.at[k], acc.at[didx.at[ch + k]], ss[k], add=True)
            for k in range(4):
                @pl.when(ch + 4 + k < NC)
                def _(k=k):
                    pltpu.make_async_copy(
                        gbuf.at[k], acc.at[didx.at[ch + k]], ss[k]).wait()
                    pltpu.async_copy(
                        h_hbm.at[sidx.at[ch + 4 + k]], gbuf.at[k], sg[k])

        for k in range(4):
            pltpu.make_async_copy(
                gbuf.at[k], acc.at[didx.at[NC - 4 + k]], ss[k]).wait()

        plsc.subcore_barrier()
        pltpu.sync_copy(acc.at[pl.ds(s * RPT, RPT)],
                        out.at[c, i, pl.ds(s * RPT, RPT)])
        _zero_acc()
        plsc.subcore_barrier()


_sc_params = pltpu.CompilerParams(use_tc_tiling_on_sc=False)

_pass0 = functools.partial(
    pl.kernel,
    _pass0_body,
    out_type=jax.ShapeDtypeStruct((2, NE, NPAD, 16), jnp.float32),
    mesh=_mesh,
    compiler_params=_sc_params,
    scratch_types=[
        pltpu.VMEM_SHARED((NPAD, 16), jnp.float32),
        pltpu.VMEM((NC, CK), jnp.int32),
        pltpu.VMEM((NC, CK), jnp.int32),
        pltpu.VMEM((2, 16, 128), jnp.float32),
        pltpu.VMEM((2, CK, 16), jnp.float32),
        pltpu.VMEM((RPT, 16), jnp.float32),
        pltpu.SemaphoreType.DMA,
        pltpu.SemaphoreType.DMA,
    ],
)()

_layer_agg = functools.partial(
    pl.kernel,
    _layer_body,
    out_type=jax.ShapeDtypeStruct((2, NE, NPAD, NH), jnp.float32),
    mesh=_mesh,
    compiler_params=_sc_params,
    scratch_types=[
        pltpu.VMEM_SHARED((NPAD, NH), jnp.float32),
        pltpu.VMEM((NC, CK), jnp.int32),
        pltpu.VMEM((NC, CK), jnp.int32),
        pltpu.VMEM((NC, CK), jnp.int32),
        pltpu.VMEM((4, CK, NH), jnp.float32),
        pltpu.VMEM((RPT // 8, NH), jnp.float32),
        pltpu.SemaphoreType.DMA,
        pltpu.SemaphoreType.DMA,
        pltpu.SemaphoreType.DMA,
        pltpu.SemaphoreType.DMA,
        pltpu.SemaphoreType.DMA,
        pltpu.SemaphoreType.DMA,
        pltpu.SemaphoreType.DMA,
        pltpu.SemaphoreType.DMA,
    ],
)()


def kernel(op_feats, tensor_feats, edge_index, W_op, b_op, W_edge, b_edge,
           W_gc, b_gc, W_fin, b_fin):
    f32 = jnp.float32

    src = edge_index[:, 0, :].reshape(NE, NW, EPW)
    dst = edge_index[:, 1, :].reshape(NE, NW, EPW)
    pad_src = jnp.broadcast_to((jnp.arange(PADN, dtype=jnp.int32) * 37) % N,
                               (NE, NW, PADN))
    pad_dst = jnp.broadcast_to(N + (jnp.arange(PADN, dtype=jnp.int32) % (NPAD - N)),
                               (NE, NW, PADN))
    pkW = jnp.concatenate([src + dst * 65536, pad_src + pad_dst * 65536],
                          axis=2).reshape(IDXR, CK)

    opf_pad = jnp.concatenate(
        [op_feats, jnp.zeros((NPAD - N, D_OP), f32)], axis=0)
    h = pl.pallas_call(
        _h0_body,
        out_shape=jax.ShapeDtypeStruct((NPAD, NH), f32),
    )(opf_pad, W_op, b_op)

    tf8 = tensor_feats.reshape(E // 8, 8 * D_T)
    W8 = jnp.einsum('jak,ikc,jbc->iab', _SEL_IN, W_edge, _SEL_OUT)
    b8 = jnp.einsum('jbc,ic->ib', _SEL_OUT, b_edge) + _CNT_COL[None, :]
    y8 = pl.pallas_call(
        _efsp_body,
        grid=(NE, NW // 4),
        in_specs=[
            pl.BlockSpec((4 * EPW8, 128), lambda i, w: (w, 0)),
            pl.BlockSpec((1, 128, 128), lambda i, w: (i, 0, 0)),
            pl.BlockSpec((1, 1, 128), lambda i, w: (i, 0, 0)),
        ],
        out_specs=pl.BlockSpec((1, 4, SLOTS8, 128), lambda i, w: (i, w, 0, 0)),
        out_shape=jax.ShapeDtypeStruct((NE, NW, SLOTS8, 128), f32),
    )(tf8, W8, b8[:, None, :])

    p0 = _pass0(y8, pkW)
    S = p0[0] + p0[1]
    cnt = S[..., EH]
    invc = 1.0 / jnp.maximum(cnt, 1.0)
    mask = (cnt > 0).astype(f32)
    M_e = S[..., :EH] * invc[..., None]

    W1 = W_gc[:, :, :NH, :]
    W2 = W_gc[:, :, NH:, :]

    BN = 2528
    for l in range(L):
        p = _layer_agg(h, pkW)
        h = pl.pallas_call(
            functools.partial(_combine_body, l == L - 1),
            grid=(NPAD // BN,),
            in_specs=[
                pl.BlockSpec((BN, NH), lambda r: (r, 0)),
                pl.BlockSpec((2, NE, BN, NH), lambda r: (0, 0, r, 0)),
                pl.BlockSpec((BN, NE), lambda r: (r, 0)),
                pl.BlockSpec((BN, NE), lambda r: (r, 0)),
                pl.BlockSpec((NE, BN, EH), lambda r: (0, r, 0)),
                pl.BlockSpec((NE, NH, NH), lambda r: (0, 0, 0)),
                pl.BlockSpec((NE, EH, NH), lambda r: (0, 0, 0)),
                pl.BlockSpec((NE, NH), lambda r: (0, 0)),
            ],
            out_specs=pl.BlockSpec((BN, NH), lambda r: (r, 0)),
            out_shape=jax.ShapeDtypeStruct((NPAD, NH), f32),
        )(h, p, invc.T, mask.T, M_e, W1[l], W2[l], b_gc[l])

    out = pl.pallas_call(
        _final_body,
        out_shape=jax.ShapeDtypeStruct((NPAD, 1), f32),
    )(h, W_fin, b_fin)
    return out[:N, 0]


# speedup vs baseline: 1.4758x; 1.0022x over previous
import functools

import jax
import jax.numpy as jnp
import numpy as np
from jax import lax
from jax.experimental import pallas as pl
from jax.experimental.pallas import tpu as pltpu
from jax.experimental.pallas import tpu_sc as plsc

N = 10000
E = 320000
NE = 5
D_OP = 128
D_T = 16
NH = 64
EH = 8
L = 6

NPAD = 10112
NW = 32
EPW = E // NW
CK = 128
NC = 80
SLOTS = NC * CK
PADN = SLOTS - EPW
RPT = NPAD // 16
EPW8 = EPW // 8
SLOTS8 = SLOTS // 8
IDXR = NE * NW * NC

_mesh = plsc.VectorSubcoreMesh(core_axis_name="c", subcore_axis_name="s")

_SEL_IN = np.zeros((8, 128, D_T), np.float32)
_SEL_OUT = np.zeros((8, 128, EH), np.float32)
_CNT_COL = np.zeros((128,), np.float32)
for _j in range(8):
    for _k in range(D_T):
        _SEL_IN[_j, 16 * _j + _k, _k] = 1.0
    for _c in range(EH):
        _SEL_OUT[_j, 16 * _j + _c, _c] = 1.0
    _CNT_COL[16 * _j + EH] = 1.0


def _elu(x):
    return jnp.where(x > 0, x, jnp.exp(jnp.minimum(x, 0.0)) - 1.0)


def _h0_body(x_ref, w_ref, b_ref, o_ref):
    o_ref[...] = _elu(
        jnp.dot(x_ref[...], w_ref[...], preferred_element_type=jnp.float32)
        + b_ref[...][None, :])


def _efsp_body(tf_ref, w_ref, b_ref, o_ref):
    y = _elu(
        jnp.dot(tf_ref[...], w_ref[0], preferred_element_type=jnp.float32)
        + b_ref[0, 0][None, :])
    zpad = jnp.zeros((SLOTS8 - EPW8, 128), jnp.float32)
    for k in range(4):
        o_ref[0, k] = jnp.concatenate(
            [y[k * EPW8:(k + 1) * EPW8], zpad], axis=0)


def _combine_body(last, h_ref, p_ref, invc_ref, mask_ref, me_ref,
                  w1_ref, w2_ref, b_ref, o_ref):
    h = h_ref[...]
    acc = jnp.zeros_like(h)
    for i in range(NE):
        t = (p_ref[0, i] + p_ref[1, i]) * invc_ref[:, i][:, None]
        acc = acc + jnp.dot(t, w1_ref[i], preferred_element_type=jnp.float32)
        acc = acc + jnp.dot(me_ref[i], w2_ref[i], preferred_element_type=jnp.float32)
        acc = acc + mask_ref[:, i][:, None] * b_ref[i][None, :]
    r = h + acc * (1.0 / NE)
    o_ref[...] = r if last else _elu(r)


def _final_body(h_ref, w_ref, b_ref, o_ref):
    o_ref[...] = jnp.dot(h_ref[...], w_ref[...],
                         preferred_element_type=jnp.float32) + b_ref[0]


def _zero_rows(buf, rows, width):
    z = jnp.zeros((16,), jnp.float32)

    @pl.loop(0, rows)
    def _(r):
        for k in range(width // 16):
            buf[r, pl.ds(k * 16, 16)] = z


def _unpack16(src, dst):
    for e in range(CK):
        dst[e, pl.ds(0, 16)] = src[e // 8, pl.ds(16 * (e % 8), 16)]


def _unpack_idx(pidx, sidx, didx):
    @pl.loop(0, NC)
    def _(r):
        for k in range(CK // 16):
            v = pidx[r, pl.ds(k * 16, 16)]
            if sidx is not None:
                sidx[r, pl.ds(k * 16, 16)] = v & 0xFFFF
            didx[r, pl.ds(k * 16, 16)] = v >> 16


def _pass0_body(efsp, pkw, out, tab, pidx, idx, ebuf, sbuf, zbuf,
                sg0, sg1, ss0, ss1):
    c = lax.axis_index("c")
    s = lax.axis_index("s")
    w = c * 16 + s
    sem_g = (sg0, sg1)
    sem_s = (ss0, ss1)
    _zero_rows(zbuf, RPT, 16)
    pltpu.sync_copy(zbuf, tab.at[pl.ds(s * RPT, RPT)])
    plsc.subcore_barrier()
    for i in range(NE):
        pltpu.sync_copy(pkw.at[pl.ds((i * NW + w) * NC, NC)], pidx)
        _unpack_idx(pidx, None, idx)
        for k in range(2):
            pltpu.async_copy(efsp.at[i, w, pl.ds(k * 16, 16)],
                             ebuf.at[k], sem_g[k])

        @pl.loop(0, NC, step=2)
        def _(ch):
            for k in range(2):
                pltpu.make_async_copy(
                    efsp.at[i, w, pl.ds((ch + k) * 16, 16)],
                    ebuf.at[k], sem_g[k]).wait()
                _unpack16(ebuf.at[k], sbuf.at[k])
                pltpu.async_copy(sbuf.at[k], tab.at[idx.at[ch + k]],
                                 sem_s[k], add=True)

            for k in range(2):
                @pl.when(ch + 2 + k < NC)
                def _(k=k):
                    pltpu.async_copy(
                        efsp.at[i, w, pl.ds((ch + 2 + k) * 16, 16)],
                        ebuf.at[k], sem_g[k])
                    pltpu.make_async_copy(
                        sbuf.at[k], tab.at[idx.at[ch + k]], sem_s[k]).wait()

        for k in range(2):
            pltpu.make_async_copy(
                sbuf.at[k], tab.at[idx.at[NC - 2 + k]], sem_s[k]).wait()

        plsc.subcore_barrier()
        pltpu.sync_copy(tab.at[pl.ds(s * RPT, RPT)],
                        out.at[c, i, pl.ds(s * RPT, RPT)])
        pltpu.sync_copy(zbuf, tab.at[pl.ds(s * RPT, RPT)])
        plsc.subcore_barrier()


def _layer_body(h_hbm, pkw, out, acc, pidx, sidx, didx, gbuf, zbuf,
                sg0, sg1, sg2, sg3, ss0, ss1, ss2, ss3):
    c = lax.axis_index("c")
    s = lax.axis_index("s")
    w = c * 16 + s
    sg = (sg0, sg1, sg2, sg3)
    ss = (ss0, ss1, ss2, ss3)
    _zero_rows(zbuf, RPT // 8, NH)

    def _zero_acc():
        for r8 in range(8):
            pltpu.sync_copy(
                zbuf, acc.at[pl.ds(s * RPT + r8 * (RPT // 8), RPT // 8)])

    _zero_acc()
    plsc.subcore_barrier()
    for i in range(NE):
        pltpu.sync_copy(pkw.at[pl.ds((i * NW + w) * NC, NC)], pidx)
        _unpack_idx(pidx, sidx, didx)
        for k in range(4):
            pltpu.async_copy(h_hbm.at[sidx.at[k]], gbuf.at[k], sg[k])

        @pl.loop(0, NC, step=4)
        def _(ch):
            for k in range(4):
                pltpu.make_async_copy(
                    h_hbm.at[sidx.at[ch + k]], gbuf.at[k], sg[k]).wait()
                pltpu.async_copy(
                    gbuf.at[k], acc.at[didx.at[ch + k]], ss[k], add=True)
            for k in range(4):
                @pl.when(ch + 4 + k < NC)
                def _(k=k):
                    pltpu.make_async_copy(
                        gbuf.at[k], acc.at[didx.at[ch + k]], ss[k]).wait()
                    pltpu.async_copy(
                        h_hbm.at[sidx.at[ch + 4 + k]], gbuf.at[k], sg[k])

        for k in range(4):
            pltpu.make_async_copy(
                gbuf.at[k], acc.at[didx.at[NC - 4 + k]], ss[k]).wait()

        plsc.subcore_barrier()
        pltpu.sync_copy(acc.at[pl.ds(s * RPT, RPT)],
                        out.at[c, i, pl.ds(s * RPT, RPT)])
        _zero_acc()
        plsc.subcore_barrier()


_sc_params = pltpu.CompilerParams(use_tc_tiling_on_sc=False)

_pass0 = functools.partial(
    pl.kernel,
    _pass0_body,
    out_type=jax.ShapeDtypeStruct((2, NE, NPAD, 16), jnp.float32),
    mesh=_mesh,
    compiler_params=_sc_params,
    scratch_types=[
        pltpu.VMEM_SHARED((NPAD, 16), jnp.float32),
        pltpu.VMEM((NC, CK), jnp.int32),
        pltpu.VMEM((NC, CK), jnp.int32),
        pltpu.VMEM((2, 16, 128), jnp.float32),
        pltpu.VMEM((2, CK, 16), jnp.float32),
        pltpu.VMEM((RPT, 16), jnp.float32),
        pltpu.SemaphoreType.DMA,
        pltpu.SemaphoreType.DMA,
        pltpu.SemaphoreType.DMA,
        pltpu.SemaphoreType.DMA,
    ],
)()

_layer_agg = functools.partial(
    pl.kernel,
    _layer_body,
    out_type=jax.ShapeDtypeStruct((2, NE, NPAD, NH), jnp.float32),
    mesh=_mesh,
    compiler_params=_sc_params,
    scratch_types=[
        pltpu.VMEM_SHARED((NPAD, NH), jnp.float32),
        pltpu.VMEM((NC, CK), jnp.int32),
        pltpu.VMEM((NC, CK), jnp.int32),
        pltpu.VMEM((NC, CK), jnp.int32),
        pltpu.VMEM((4, CK, NH), jnp.float32),
        pltpu.VMEM((RPT // 8, NH), jnp.float32),
        pltpu.SemaphoreType.DMA,
        pltpu.SemaphoreType.DMA,
        pltpu.SemaphoreType.DMA,
        pltpu.SemaphoreType.DMA,
        pltpu.SemaphoreType.DMA,
        pltpu.SemaphoreType.DMA,
        pltpu.SemaphoreType.DMA,
        pltpu.SemaphoreType.DMA,
    ],
)()


def kernel(op_feats, tensor_feats, edge_index, W_op, b_op, W_edge, b_edge,
           W_gc, b_gc, W_fin, b_fin):
    f32 = jnp.float32

    src = edge_index[:, 0, :].reshape(NE, NW, EPW)
    dst = edge_index[:, 1, :].reshape(NE, NW, EPW)
    pad_src = jnp.broadcast_to((jnp.arange(PADN, dtype=jnp.int32) * 37) % N,
                               (NE, NW, PADN))
    pad_dst = jnp.broadcast_to(N + (jnp.arange(PADN, dtype=jnp.int32) % (NPAD - N)),
                               (NE, NW, PADN))
    pkW = jnp.concatenate([src + dst * 65536, pad_src + pad_dst * 65536],
                          axis=2).reshape(IDXR, CK)

    opf_pad = jnp.concatenate(
        [op_feats, jnp.zeros((NPAD - N, D_OP), f32)], axis=0)
    h = pl.pallas_call(
        _h0_body,
        out_shape=jax.ShapeDtypeStruct((NPAD, NH), f32),
    )(opf_pad, W_op, b_op)

    tf8 = tensor_feats.reshape(E // 8, 8 * D_T)
    W8 = jnp.einsum('jak,ikc,jbc->iab', _SEL_IN, W_edge, _SEL_OUT)
    b8 = jnp.einsum('jbc,ic->ib', _SEL_OUT, b_edge) + _CNT_COL[None, :]
    y8 = pl.pallas_call(
        _efsp_body,
        grid=(NE, NW // 4),
        in_specs=[
            pl.BlockSpec((4 * EPW8, 128), lambda i, w: (w, 0)),
            pl.BlockSpec((1, 128, 128), lambda i, w: (i, 0, 0)),
            pl.BlockSpec((1, 1, 128), lambda i, w: (i, 0, 0)),
        ],
        out_specs=pl.BlockSpec((1, 4, SLOTS8, 128), lambda i, w: (i, w, 0, 0)),
        out_shape=jax.ShapeDtypeStruct((NE, NW, SLOTS8, 128), f32),
    )(tf8, W8, b8[:, None, :])

    p0 = _pass0(y8, pkW)
    S = p0[0] + p0[1]
    cnt = S[..., EH]
    invc = 1.0 / jnp.maximum(cnt, 1.0)
    mask = (cnt > 0).astype(f32)
    M_e = S[..., :EH] * invc[..., None]

    W1 = W_gc[:, :, :NH, :]
    W2 = W_gc[:, :, NH:, :]

    BN = 2528
    for l in range(L):
        p = _layer_agg(h, pkW)
        h = pl.pallas_call(
            functools.partial(_combine_body, l == L - 1),
            grid=(NPAD // BN,),
            in_specs=[
                pl.BlockSpec((BN, NH), lambda r: (r, 0)),
                pl.BlockSpec((2, NE, BN, NH), lambda r: (0, 0, r, 0)),
                pl.BlockSpec((BN, NE), lambda r: (r, 0)),
                pl.BlockSpec((BN, NE), lambda r: (r, 0)),
                pl.BlockSpec((NE, BN, EH), lambda r: (0, r, 0)),
                pl.BlockSpec((NE, NH, NH), lambda r: (0, 0, 0)),
                pl.BlockSpec((NE, EH, NH), lambda r: (0, 0, 0)),
                pl.BlockSpec((NE, NH), lambda r: (0, 0)),
            ],
            out_specs=pl.BlockSpec((BN, NH), lambda r: (r, 0)),
            out_shape=jax.ShapeDtypeStruct((NPAD, NH), f32),
        )(h, p, invc.T, mask.T, M_e, W1[l], W2[l], b_gc[l])

    out = pl.pallas_call(
        _final_body,
        out_shape=jax.ShapeDtypeStruct((NPAD, 1), f32),
    )(h, W_fin, b_fin)
    return out[:N, 0]


# speedup vs baseline: 1.4839x; 1.0055x over previous
import functools

import jax
import jax.numpy as jnp
import numpy as np
from jax import lax
from jax.experimental import pallas as pl
from jax.experimental.pallas import tpu as pltpu
from jax.experimental.pallas import tpu_sc as plsc

N = 10000
E = 320000
NE = 5
D_OP = 128
D_T = 16
NH = 64
EH = 8
L = 6

NPAD = 10112
NW = 32
EPW = E // NW
CK = 128
NC = 80
SLOTS = NC * CK
PADN = SLOTS - EPW
RPT = NPAD // 16
EPW8 = EPW // 8
SLOTS8 = SLOTS // 8
IDXR = NE * NW * NC

_mesh = plsc.VectorSubcoreMesh(core_axis_name="c", subcore_axis_name="s")

_SEL_IN = np.zeros((8, 128, D_T), np.float32)
_SEL_OUT = np.zeros((8, 128, EH), np.float32)
_CNT_COL = np.zeros((128,), np.float32)
for _j in range(8):
    for _k in range(D_T):
        _SEL_IN[_j, 16 * _j + _k, _k] = 1.0
    for _c in range(EH):
        _SEL_OUT[_j, 16 * _j + _c, _c] = 1.0
    _CNT_COL[16 * _j + EH] = 1.0


def _elu(x):
    return jnp.where(x > 0, x, jnp.exp(jnp.minimum(x, 0.0)) - 1.0)


def _h0_body(x_ref, w_ref, b_ref, o_ref):
    o_ref[...] = _elu(
        jnp.dot(x_ref[...], w_ref[...], preferred_element_type=jnp.float32)
        + b_ref[...][None, :])


def _efsp_body(tf_ref, w_ref, b_ref, o_ref):
    y = _elu(
        jnp.dot(tf_ref[...], w_ref[0], preferred_element_type=jnp.float32)
        + b_ref[0, 0][None, :])
    zpad = jnp.zeros((SLOTS8 - EPW8, 128), jnp.float32)
    for k in range(4):
        o_ref[0, k] = jnp.concatenate(
            [y[k * EPW8:(k + 1) * EPW8], zpad], axis=0)


def _combine_body(last, h_ref, p_ref, invc_ref, mask_ref, me_ref,
                  w1_ref, w2_ref, b_ref, o_ref):
    h = h_ref[...]
    acc = jnp.zeros_like(h)
    for i in range(NE):
        t = (p_ref[0, i] + p_ref[1, i]) * invc_ref[:, i][:, None]
        acc = acc + jnp.dot(t, w1_ref[i], preferred_element_type=jnp.float32)
        acc = acc + jnp.dot(me_ref[i], w2_ref[i], preferred_element_type=jnp.float32)
        acc = acc + mask_ref[:, i][:, None] * b_ref[i][None, :]
    r = h + acc * (1.0 / NE)
    o_ref[...] = r if last else _elu(r)


def _final_body(h_ref, w_ref, b_ref, o_ref):
    o_ref[...] = jnp.dot(h_ref[...], w_ref[...],
                         preferred_element_type=jnp.float32) + b_ref[0]


def _zero_rows(buf, rows, width):
    z = jnp.zeros((16,), jnp.float32)

    @pl.loop(0, rows)
    def _(r):
        for k in range(width // 16):
            buf[r, pl.ds(k * 16, 16)] = z


def _unpack16(src, dst):
    for e in range(CK):
        dst[e, pl.ds(0, 16)] = src[e // 8, pl.ds(16 * (e % 8), 16)]


def _unpack_idx(pidx, sidx, didx):
    @pl.loop(0, NC)
    def _(r):
        for k in range(CK // 16):
            v = pidx[r, pl.ds(k * 16, 16)]
            if sidx is not None:
                sidx[r, pl.ds(k * 16, 16)] = v & 0xFFFF
            didx[r, pl.ds(k * 16, 16)] = v >> 16


def _pass0_body(efsp, pkw, out, tab, pidx, idx, ebuf, sbuf, zbuf,
                sg0, sg1, ss0, ss1):
    c = lax.axis_index("c")
    s = lax.axis_index("s")
    w = c * 16 + s
    sem_g = (sg0, sg1)
    sem_s = (ss0, ss1)
    _zero_rows(zbuf, RPT, 16)
    pltpu.sync_copy(zbuf, tab.at[pl.ds(s * RPT, RPT)])
    plsc.subcore_barrier()
    for i in range(NE):
        pltpu.sync_copy(pkw.at[pl.ds((i * NW + w) * NC, NC)], pidx)
        _unpack_idx(pidx, None, idx)
        for k in range(2):
            pltpu.async_copy(efsp.at[i, w, pl.ds(k * 16, 16)],
                             ebuf.at[k], sem_g[k])

        @pl.loop(0, NC, step=2)
        def _(ch):
            for k in range(2):
                pltpu.make_async_copy(
                    efsp.at[i, w, pl.ds((ch + k) * 16, 16)],
                    ebuf.at[k], sem_g[k]).wait()
                _unpack16(ebuf.at[k], sbuf.at[k])
                pltpu.async_copy(sbuf.at[k], tab.at[idx.at[ch + k]],
                                 sem_s[k], add=True)

            for k in range(2):
                @pl.when(ch + 2 + k < NC)
                def _(k=k):
                    pltpu.async_copy(
                        efsp.at[i, w, pl.ds((ch + 2 + k) * 16, 16)],
                        ebuf.at[k], sem_g[k])
                    pltpu.make_async_copy(
                        sbuf.at[k], tab.at[idx.at[ch + k]], sem_s[k]).wait()

        for k in range(2):
            pltpu.make_async_copy(
                sbuf.at[k], tab.at[idx.at[NC - 2 + k]], sem_s[k]).wait()

        plsc.subcore_barrier()
        pltpu.sync_copy(tab.at[pl.ds(s * RPT, RPT)],
                        out.at[c, i, pl.ds(s * RPT, RPT)])
        pltpu.sync_copy(zbuf, tab.at[pl.ds(s * RPT, RPT)])
        plsc.subcore_barrier()


def _layer_body(h_hbm, pkw, out, acc, pidx, sidx, didx, gbuf, zbuf,
                sg0, sg1, sg2, sg3, ss0, ss1, ss2, ss3):
    c = lax.axis_index("c")
    s = lax.axis_index("s")
    w = c * 16 + s
    sg = (sg0, sg1, sg2, sg3)
    ss = (ss0, ss1, ss2, ss3)
    _zero_rows(zbuf, RPT // 8, NH)

    def _zero_acc():
        for r8 in range(8):
            pltpu.sync_copy(
                zbuf, acc.at[pl.ds(s * RPT + r8 * (RPT // 8), RPT // 8)])

    _zero_acc()
    plsc.subcore_barrier()
    for i in range(NE):
        pltpu.sync_copy(pkw.at[pl.ds((i * NW + w) * NC, NC)], pidx)
        _unpack_idx(pidx, sidx, didx)
        for k in range(4):
            pltpu.async_copy(h_hbm.at[sidx.at[k]], gbuf.at[k], sg[k])

        @pl.loop(0, NC, step=4)
        def _(ch):
            for k in range(4):
                pltpu.make_async_copy(
                    h_hbm.at[sidx.at[ch + k]], gbuf.at[k], sg[k]).wait()
                pltpu.async_copy(
                    gbuf.at[k], acc.at[didx.at[ch + k]], ss[k], add=True)
            for k in range(4):
                @pl.when(ch + 4 + k < NC)
                def _(k=k):
                    pltpu.make_async_copy(
                        gbuf.at[k], acc.at[didx.at[ch + k]], ss[k]).wait()
                    pltpu.async_copy(
                        h_hbm.at[sidx.at[ch + 4 + k]], gbuf.at[k], sg[k])

        for k in range(4):
            pltpu.make_async_copy(
                gbuf.at[k], acc.at[didx.at[NC - 4 + k]], ss[k]).wait()

        plsc.subcore_barrier()
        pltpu.sync_copy(acc.at[pl.ds(s * RPT, RPT)],
                        out.at[c, i, pl.ds(s * RPT, RPT)])
        _zero_acc()
        plsc.subcore_barrier()


_sc_params = pltpu.CompilerParams(use_tc_tiling_on_sc=False)

_pass0 = functools.partial(
    pl.kernel,
    _pass0_body,
    out_type=jax.ShapeDtypeStruct((2, NE, NPAD, 16), jnp.float32),
    mesh=_mesh,
    compiler_params=_sc_params,
    scratch_types=[
        pltpu.VMEM_SHARED((NPAD, 16), jnp.float32),
        pltpu.VMEM((NC, CK), jnp.int32),
        pltpu.VMEM((NC, CK), jnp.int32),
        pltpu.VMEM((2, 16, 128), jnp.float32),
        pltpu.VMEM((2, CK, 16), jnp.float32),
        pltpu.VMEM((RPT, 16), jnp.float32),
        pltpu.SemaphoreType.DMA,
        pltpu.SemaphoreType.DMA,
        pltpu.SemaphoreType.DMA,
        pltpu.SemaphoreType.DMA,
    ],
)()

_layer_agg = functools.partial(
    pl.kernel,
    _layer_body,
    out_type=jax.ShapeDtypeStruct((2, NE, NPAD, NH), jnp.float32),
    mesh=_mesh,
    compiler_params=_sc_params,
    scratch_types=[
        pltpu.VMEM_SHARED((NPAD, NH), jnp.float32),
        pltpu.VMEM((NC, CK), jnp.int32),
        pltpu.VMEM((NC, CK), jnp.int32),
        pltpu.VMEM((NC, CK), jnp.int32),
        pltpu.VMEM((4, CK, NH), jnp.float32),
        pltpu.VMEM((RPT // 8, NH), jnp.float32),
        pltpu.SemaphoreType.DMA,
        pltpu.SemaphoreType.DMA,
        pltpu.SemaphoreType.DMA,
        pltpu.SemaphoreType.DMA,
        pltpu.SemaphoreType.DMA,
        pltpu.SemaphoreType.DMA,
        pltpu.SemaphoreType.DMA,
        pltpu.SemaphoreType.DMA,
    ],
)()


def kernel(op_feats, tensor_feats, edge_index, W_op, b_op, W_edge, b_edge,
           W_gc, b_gc, W_fin, b_fin):
    f32 = jnp.float32

    src = edge_index[:, 0, :].reshape(NE, NW, EPW)
    dst = edge_index[:, 1, :].reshape(NE, NW, EPW)
    pad_src = jnp.broadcast_to((jnp.arange(PADN, dtype=jnp.int32) * 37) % N,
                               (NE, NW, PADN))
    pad_dst = jnp.broadcast_to(N + (jnp.arange(PADN, dtype=jnp.int32) % (NPAD - N)),
                               (NE, NW, PADN))
    pkW = jnp.concatenate([src + dst * 65536, pad_src + pad_dst * 65536],
                          axis=2).reshape(IDXR, CK)

    opf_pad = jnp.concatenate(
        [op_feats, jnp.zeros((NPAD - N, D_OP), f32)], axis=0)
    h = pl.pallas_call(
        _h0_body,
        out_shape=jax.ShapeDtypeStruct((NPAD, NH), f32),
    )(opf_pad, W_op, b_op)

    tf8 = tensor_feats.reshape(E // 8, 8 * D_T)
    W8 = jnp.einsum('jak,ikc,jbc->iab', _SEL_IN, W_edge, _SEL_OUT)
    b8 = jnp.einsum('jbc,ic->ib', _SEL_OUT, b_edge) + _CNT_COL[None, :]
    y8 = pl.pallas_call(
        _efsp_body,
        grid=(NE, NW // 4),
        in_specs=[
            pl.BlockSpec((4 * EPW8, 128), lambda i, w: (w, 0)),
            pl.BlockSpec((1, 128, 128), lambda i, w: (i, 0, 0)),
            pl.BlockSpec((1, 1, 128), lambda i, w: (i, 0, 0)),
        ],
        out_specs=pl.BlockSpec((1, 4, SLOTS8, 128), lambda i, w: (i, w, 0, 0)),
        out_shape=jax.ShapeDtypeStruct((NE, NW, SLOTS8, 128), f32),
    )(tf8, W8, b8[:, None, :])

    p0 = _pass0(y8, pkW)
    S = p0[0] + p0[1]
    cnt = S[..., EH]
    invc = 1.0 / jnp.maximum(cnt, 1.0)
    mask = (cnt > 0).astype(f32)
    M_e = S[..., :EH] * invc[..., None]

    W1 = W_gc[:, :, :NH, :]
    W2 = W_gc[:, :, NH:, :]

    BN = 1264
    for l in range(L):
        p = _layer_agg(h, pkW)
        h = pl.pallas_call(
            functools.partial(_combine_body, l == L - 1),
            grid=(NPAD // BN,),
            in_specs=[
                pl.BlockSpec((BN, NH), lambda r: (r, 0)),
                pl.BlockSpec((2, NE, BN, NH), lambda r: (0, 0, r, 0)),
                pl.BlockSpec((BN, NE), lambda r: (r, 0)),
                pl.BlockSpec((BN, NE), lambda r: (r, 0)),
                pl.BlockSpec((NE, BN, EH), lambda r: (0, r, 0)),
                pl.BlockSpec((NE, NH, NH), lambda r: (0, 0, 0)),
                pl.BlockSpec((NE, EH, NH), lambda r: (0, 0, 0)),
                pl.BlockSpec((NE, NH), lambda r: (0, 0)),
            ],
            out_specs=pl.BlockSpec((BN, NH), lambda r: (r, 0)),
            out_shape=jax.ShapeDtypeStruct((NPAD, NH), f32),
        )(h, p, invc.T, mask.T, M_e, W1[l], W2[l], b_gc[l])

    out = pl.pallas_call(
        _final_body,
        out_shape=jax.ShapeDtypeStruct((NPAD, 1), f32),
    )(h, W_fin, b_fin)
    return out[:N, 0]
